# baseline (speedup 1.0000x reference)
import functools

import jax
import jax.numpy as jnp
from jax import lax
from jax.experimental import pallas as pl
from jax.experimental.pallas import tpu as pltpu
from jax.experimental.pallas import tpu_sc as plsc

N = 10000
NP = 10240
D = 256
OUT = 256
HID = 128
E = 160000

NC = 2
NS = 16
NQ = 4
QR = NP // NQ
RPT = QR // NS
EPT = E // NS
GCH = 128
STRIP = 2000
PADROW = NP - 8
KCAP = 10368
TRASH = KCAP - 16
DEGSZ = QR + 16

BM = 512


def _sc_segsum_body(xn, srcl, dstl, agg_o, deg_o,
                    sbuf0, sbuf1, dbuf0, dbuf1, kpack, ridx, sidx0, sidx1,
                    didx0, didx1, rows0, rows1, degt, acc, sdeg, gsem, esem):
    c = lax.axis_index("c")
    s = lax.axis_index("s")
    zero16f = jnp.zeros((16,), jnp.float32)
    ones16f = jnp.ones((16,), jnp.float32)
    iota16 = lax.iota(jnp.int32, 16)
    sbuf2 = (sbuf0, sbuf1)
    dbuf2 = (dbuf0, dbuf1)
    rows2 = (rows0, rows1)
    sidx2 = (sidx0, sidx1)
    didx2 = (didx0, didx1)

    ebase = s * EPT
    base = s * RPT
    nstrip = EPT // STRIP

    def _strip_copies(t, par):
        return (pltpu.make_async_copy(
                    srcl.at[pl.ds(ebase + t * STRIP, STRIP)], sbuf2[par], esem),
                pltpu.make_async_copy(
                    dstl.at[pl.ds(ebase + t * STRIP, STRIP)], dbuf2[par], esem))

    for p in range(2):
        q = 2 * p + c
        lo = q * QR

        def _zd(i, _):
            degt[pl.ds(i * 16, 16)] = zero16f
            return 0
        lax.fori_loop(0, DEGSZ // 16, _zd, 0)

        def _zr(i, _):
            def _zl(k, _):
                rows0[i, 0, pl.ds(k * 16, 16)] = zero16f
                rows0[i, 1, pl.ds(k * 16, 16)] = zero16f
                return 0
            return lax.fori_loop(0, 8, _zl, 0)
        lax.fori_loop(0, 32, _zr, 0)
        z32 = rows0.at[pl.ds(0, 32)]
        for off in range(0, RPT, 32):
            pltpu.sync_copy(z32, acc.at[pl.ds(base + off, 32)])

        @pl.when(s == 0)
        def _():
            pltpu.sync_copy(degt.at[pl.ds(0, QR)], sdeg)

        plsc.subcore_barrier()

        for cp in _strip_copies(0, 0):
            cp.start()
        cnt = jnp.int32(0)
        for t in range(nstrip):
            par = t % 2
            for cp in _strip_copies(t, par):
                cp.wait()
            if t + 1 < nstrip:
                for cp in _strip_copies(t + 1, 1 - par):
                    cp.start()
            sb, db = sbuf2[par], dbuf2[par]

            def _filt(j, cnt):
                sv = sb[pl.ds(j * 16, 16)]
                dv = db[pl.ds(j * 16, 16)]
                dl = dv - lo
                m = (dl >= 0) & (dl < QR)
                mi = jnp.where(m, 1, 0).astype(jnp.int32)
                inc = plsc.cumsum(mi)
                widx = jnp.where(m, cnt + inc - mi, TRASH + iota16)
                plsc.store_scatter(kpack, [widx], (sv << 12) | (dl & 4095))
                plsc.addupdate_scatter(
                    degt, [jnp.where(m, dl, QR + iota16)], ones16f)
                return cnt + inc[15]
            cnt = lax.fori_loop(0, STRIP // 16, _filt, cnt)

        padpk = jnp.full((16,), PADROW << 12, jnp.int32)
        for k in range(2 * GCH // 16):
            kpack[pl.ds(cnt + k * 16, 16)] = padpk
        ngc2 = jnp.maximum((cnt + 2 * GCH - 1) // (2 * GCH) * 2, 2)

        def _issue(g, b):
            for k in range(GCH // 16):
                pk = kpack[pl.ds(g * GCH + k * 16, 16)]
                sidx2[b][pl.ds(k * 16, 16)] = pk >> 12
                didx2[b][pl.ds(k * 16, 16)] = pk & 4095
            pltpu.async_copy(xn.at[sidx2[b]], rows2[b], gsem)

        _issue(0, 0)

        def _pair(t, _):
            for b in (0, 1):
                g = 2 * t + b
                pltpu.make_async_copy(
                    xn.at[sidx2[b]], rows2[b], gsem).wait()

                @pl.when(g + 1 < ngc2)
                def _():
                    _issue(g + 1, 1 - b)
                pltpu.sync_copy(rows2[b], acc.at[didx2[b]], add=True)
            return 0
        lax.fori_loop(0, ngc2 // 2, _pair, 0)

        def _dr(g, _):
            gb = g * GCH
            for k in range(GCH // 16):
                ridx[pl.ds(k * 16, 16)] = iota16 + (gb + k * 16)
            pltpu.sync_copy(degt.at[pl.ds(gb, GCH)], sdeg.at[ridx], add=True)
            return 0
        lax.fori_loop(0, QR // GCH, _dr, 0)

        plsc.subcore_barrier()

        pltpu.sync_copy(acc.at[pl.ds(base, RPT)],
                        agg_o.at[pl.ds(lo + base, RPT)])

        @pl.when(s == 0)
        def _():
            pltpu.sync_copy(sdeg, deg_o.at[pl.ds(lo, QR)])

        plsc.subcore_barrier()


@functools.lru_cache(maxsize=1)
def _build_sc_segsum():
  return functools.partial(
    pl.kernel,
    out_type=[
        jax.ShapeDtypeStruct((NP, 2, OUT // 2), jnp.float32),
        jax.ShapeDtypeStruct((NP,), jnp.float32),
    ],
    mesh=plsc.VectorSubcoreMesh(
        core_axis_name="c", subcore_axis_name="s",
        num_cores=NC, num_subcores=NS),
    compiler_params=pltpu.CompilerParams(needs_layout_passes=False),
    scratch_types=[
        pltpu.VMEM((STRIP,), jnp.int32),
        pltpu.VMEM((STRIP,), jnp.int32),
        pltpu.VMEM((STRIP,), jnp.int32),
        pltpu.VMEM((STRIP,), jnp.int32),
        pltpu.VMEM((KCAP,), jnp.int32),
        pltpu.VMEM((GCH,), jnp.int32),
        pltpu.VMEM((GCH,), jnp.int32),
        pltpu.VMEM((GCH,), jnp.int32),
        pltpu.VMEM((GCH,), jnp.int32),
        pltpu.VMEM((GCH,), jnp.int32),
        pltpu.VMEM((GCH, 2, OUT // 2), jnp.float32),
        pltpu.VMEM((GCH, 2, OUT // 2), jnp.float32),
        pltpu.VMEM((DEGSZ,), jnp.float32),
        pltpu.VMEM_SHARED((QR, 2, OUT // 2), jnp.float32),
        pltpu.VMEM_SHARED((QR,), jnp.float32),
        pltpu.SemaphoreType.DMA,
        pltpu.SemaphoreType.DMA,
    ],
  )(_sc_segsum_body)


def _segsum(xn, src, dst):
    agg, deg = _build_sc_segsum()(xn.reshape(NP, 2, OUT // 2), src, dst)
    return agg.reshape(NP, OUT), deg.reshape(NP, 1)


def _dot(a, b):
    return lax.dot_general(a, b, (((1,), (0,)), ((), ())),
                           preferred_element_type=jnp.float32)


def _row_mask(i):
    r = i * BM + lax.broadcasted_iota(jnp.int32, (BM, 1), 0)
    return r < N


def _full(shape):
    return pl.BlockSpec(shape, lambda i: (0,) * len(shape))


_ROW = pl.BlockSpec((BM, D), lambda i: (i, 0))
_DEGB = pl.BlockSpec((BM, 1), lambda i: (i, 0))


def _t1_body(h, ws0, wn0, b0, ws1, wn1, b1, ts0_o, tn0_o, ts1_o, tn1_o):
    x = h[...]
    ts0_o[...] = _dot(x, ws0[...]) + b0[...]
    tn0_o[...] = _dot(x, wn0[...])
    ts1_o[...] = _dot(x, ws1[...]) + b1[...]
    tn1_o[...] = _dot(x, wn1[...])


def _t1(h, ws0, wn0, b0, ws1, wn1, b1):
    return pl.pallas_call(
        _t1_body,
        grid=(NP // BM,),
        in_specs=[_ROW, _full((D, OUT)), _full((D, OUT)), _full((1, OUT)),
                  _full((D, OUT)), _full((D, OUT)), _full((1, OUT))],
        out_specs=[pl.BlockSpec((BM, OUT), lambda i: (i, 0))] * 4,
        out_shape=[jax.ShapeDtypeStruct((NP, OUT), jnp.float32)] * 4,
    )(h, ws0, wn0, b0.reshape(1, OUT), ws1, wn1, b1.reshape(1, OUT))


def _elu_norm(ts_ref, agg_ref, deg_ref):
    rdeg = 1.0 / jnp.maximum(deg_ref[...], 1.0)
    x = ts_ref[...] + agg_ref[...] * rdeg
    return jnp.where(x > 0, x, jnp.exp(jnp.minimum(x, 0.0)) - 1.0)


def _t2_body(ts_ref, agg_ref, deg_ref, ws, wn, b, ts_o, tn_o):
    e = _elu_norm(ts_ref, agg_ref, deg_ref)
    ts_o[...] = _dot(e, ws[...]) + b[...]
    tn_o[...] = jnp.where(_row_mask(pl.program_id(0)), _dot(e, wn[...]), 0.0)


_EIN = [_ROW, pl.BlockSpec((BM, OUT), lambda i: (i, 0)), _DEGB]


def _t2(ts, agg, deg, ws, wn, b):
    return pl.pallas_call(
        _t2_body,
        grid=(NP // BM,),
        in_specs=_EIN + [_full((D, OUT)), _full((D, OUT)), _full((1, OUT))],
        out_specs=[pl.BlockSpec((BM, OUT), lambda i: (i, 0))] * 2,
        out_shape=[jax.ShapeDtypeStruct((NP, OUT), jnp.float32)] * 2,
    )(ts, agg, deg, ws, wn, b.reshape(1, OUT))


def _t3_body(ts0, agg0, deg0, ts1, agg1, deg1, wa1, ba1, wa2, s_o):
    i = pl.program_id(0)

    @pl.when(i == 0)
    def _():
        s_o[...] = jnp.zeros((8, 128), jnp.float32)

    mask = _row_mask(i)
    w = []
    for ts_ref, agg_ref, deg_ref in ((ts0, agg0, deg0), (ts1, agg1, deg1)):
        e = _elu_norm(ts_ref, agg_ref, deg_ref)
        t = jnp.tanh(_dot(e, wa1[...]) + ba1[...])
        w.append(jnp.sum(jnp.where(mask, t * wa2[...], 0.0)))
    r = lax.broadcasted_iota(jnp.int32, (8, 128), 0)
    col = lax.broadcasted_iota(jnp.int32, (8, 128), 1)
    upd = jnp.where((r == 0) & (col == 0), w[0], 0.0) + \
          jnp.where((r == 0) & (col == 1), w[1], 0.0)
    s_o[...] += upd


def _t3(ts0, agg0, deg0, ts1, agg1, deg1, wa1, ba1, wa2):
    return pl.pallas_call(
        _t3_body,
        grid=(NP // BM,),
        in_specs=_EIN + _EIN +
                 [_full((OUT, HID)), _full((1, HID)), _full((1, HID))],
        out_specs=pl.BlockSpec((8, 128), lambda i: (0, 0)),
        out_shape=jax.ShapeDtypeStruct((8, 128), jnp.float32),
    )(ts0, agg0, deg0, ts1, agg1, deg1,
      wa1, ba1.reshape(1, HID), wa2.reshape(1, HID))


def _t4_body(ts0, agg0, deg0, ts1, agg1, deg1, s_ref, out_o):
    w0 = s_ref[0, 0] / N
    w1 = s_ref[0, 1] / N
    m = jnp.maximum(w0, w1)
    x0 = jnp.exp(w0 - m)
    x1 = jnp.exp(w1 - m)
    beta0 = x0 / (x0 + x1)
    e0 = _elu_norm(ts0, agg0, deg0)
    e1 = _elu_norm(ts1, agg1, deg1)
    out_o[...] = beta0 * e0 + (1.0 - beta0) * e1


def _t4(ts0, agg0, deg0, ts1, agg1, deg1, s):
    return pl.pallas_call(
        _t4_body,
        grid=(NP // BM,),
        in_specs=_EIN + _EIN + [_full((8, 128))],
        out_specs=pl.BlockSpec((BM, OUT), lambda i: (i, 0)),
        out_shape=jax.ShapeDtypeStruct((NP, OUT), jnp.float32),
    )(ts0, agg0, deg0, ts1, agg1, deg1, s)


def kernel(h, edge_index_0, edge_index_1,
           W_self_00, W_neigh_00, b_00, W_self_01, W_neigh_01, b_01,
           W_self_10, W_neigh_10, b_10, W_self_11, W_neigh_11, b_11,
           Wa1, ba1, Wa2):
    src0 = edge_index_0[0].astype(jnp.int32)
    dst0 = edge_index_0[1].astype(jnp.int32)
    src1 = edge_index_1[0].astype(jnp.int32)
    dst1 = edge_index_1[1].astype(jnp.int32)

    hp = jnp.zeros((NP, D), jnp.float32).at[:N].set(h.astype(jnp.float32))

    ts00, tn00, ts10, tn10 = _t1(hp, W_self_00, W_neigh_00, b_00,
                                 W_self_10, W_neigh_10, b_10)
    agg00, deg0 = _segsum(tn00, src0, dst0)
    agg10, deg1 = _segsum(tn10, src1, dst1)

    ts01, tn01 = _t2(ts00, agg00, deg0, W_self_01, W_neigh_01, b_01)
    agg01, _ = _segsum(tn01, src0, dst0)
    ts11, tn11 = _t2(ts10, agg10, deg1, W_self_11, W_neigh_11, b_11)
    agg11, _ = _segsum(tn11, src1, dst1)

    s = _t3(ts01, agg01, deg0, ts11, agg11, deg1, Wa1, ba1, Wa2)
    out = _t4(ts01, agg01, deg0, ts11, agg11, deg1, s)
    return out[:N]


# speedup vs baseline: 1.1078x; 1.1078x over previous
import functools

import jax
import jax.numpy as jnp
from jax import lax
from jax.experimental import pallas as pl
from jax.experimental.pallas import tpu as pltpu
from jax.experimental.pallas import tpu_sc as plsc

N = 10000
NP = 10240
D = 256
OUT = 256
HID = 128
E = 160000

NC = 2
NS = 16
NQ = 4
QR = NP // NQ
RPT = QR // NS
EPT = E // NS
GCH = 96
DRC = 128
STRIP = 2000
PADROW = NP - 8
KCAP = 10368
TRASH = KCAP - 16
DEGSZ = QR + 16

BM = 512


def _sc_segsum_body(xn, srcl, dstl, agg_o, deg_o,
                    src_buf, dst_buf, kpack, ridx, sidx0, sidx1,
                    didx0, didx1, rows0, rows1, degt, acc, sdeg, gsem):
    c = lax.axis_index("c")
    s = lax.axis_index("s")
    zero16f = jnp.zeros((16,), jnp.float32)
    ones16f = jnp.ones((16,), jnp.float32)
    iota16 = lax.iota(jnp.int32, 16)
    rows2 = (rows0, rows1)
    sidx2 = (sidx0, sidx1)
    didx2 = (didx0, didx1)

    ebase = s * EPT
    base = s * RPT
    pltpu.sync_copy(srcl.at[pl.ds(ebase, EPT)], src_buf)
    pltpu.sync_copy(dstl.at[pl.ds(ebase, EPT)], dst_buf)

    for p in range(2):
        q = 2 * p + c
        lo = q * QR

        def _zd(i, _):
            degt[pl.ds(i * 16, 16)] = zero16f
            return 0
        lax.fori_loop(0, DEGSZ // 16, _zd, 0)

        def _zr(i, _):
            def _zl(k, _):
                rows0[i, 0, pl.ds(k * 16, 16)] = zero16f
                rows0[i, 1, pl.ds(k * 16, 16)] = zero16f
                return 0
            return lax.fori_loop(0, 8, _zl, 0)
        lax.fori_loop(0, 32, _zr, 0)
        z32 = rows0.at[pl.ds(0, 32)]
        for off in range(0, RPT, 32):
            pltpu.sync_copy(z32, acc.at[pl.ds(base + off, 32)])

        @pl.when(s == 0)
        def _():
            pltpu.sync_copy(degt.at[pl.ds(0, QR)], sdeg)

        plsc.subcore_barrier()

        def _filt(j, cnt):
            sv = src_buf[pl.ds(j * 16, 16)]
            dv = dst_buf[pl.ds(j * 16, 16)]
            dl = dv - lo
            m = (dl >= 0) & (dl < QR)
            mi = jnp.where(m, 1, 0).astype(jnp.int32)
            inc = plsc.cumsum(mi)
            widx = jnp.where(m, cnt + inc - mi, TRASH + iota16)
            plsc.store_scatter(kpack, [widx], (sv << 12) | (dl & 4095))
            plsc.addupdate_scatter(
                degt, [jnp.where(m, dl, QR + iota16)], ones16f)
            return cnt + inc[15]
        cnt = lax.fori_loop(0, EPT // 16, _filt, jnp.int32(0))

        padpk = jnp.full((16,), PADROW << 12, jnp.int32)
        for k in range(2 * GCH // 16):
            kpack[pl.ds(cnt + k * 16, 16)] = padpk
        ngc2 = jnp.maximum((cnt + 2 * GCH - 1) // (2 * GCH) * 2, 2)

        def _issue(g, b):
            for k in range(GCH // 16):
                pk = kpack[pl.ds(g * GCH + k * 16, 16)]
                sidx2[b][pl.ds(k * 16, 16)] = pk >> 12
                didx2[b][pl.ds(k * 16, 16)] = pk & 4095
            pltpu.async_copy(xn.at[sidx2[b]], rows2[b], gsem)

        _issue(0, 0)

        def _pair(t, _):
            for b in (0, 1):
                g = 2 * t + b
                pltpu.make_async_copy(
                    xn.at[sidx2[b]], rows2[b], gsem).wait()

                @pl.when(g + 1 < ngc2)
                def _():
                    _issue(g + 1, 1 - b)
                pltpu.sync_copy(rows2[b], acc.at[didx2[b]], add=True)
            return 0
        lax.fori_loop(0, ngc2 // 2, _pair, 0)

        def _dr(g, _):
            gb = g * DRC
            for k in range(DRC // 16):
                ridx[pl.ds(k * 16, 16)] = iota16 + (gb + k * 16)
            pltpu.sync_copy(degt.at[pl.ds(gb, DRC)], sdeg.at[ridx], add=True)
            return 0
        lax.fori_loop(0, QR // DRC, _dr, 0)

        plsc.subcore_barrier()

        pltpu.sync_copy(acc.at[pl.ds(base, RPT)],
                        agg_o.at[pl.ds(lo + base, RPT)])

        @pl.when(s == 0)
        def _():
            pltpu.sync_copy(sdeg, deg_o.at[pl.ds(lo, QR)])

        plsc.subcore_barrier()


@functools.lru_cache(maxsize=1)
def _build_sc_segsum():
  return functools.partial(
    pl.kernel,
    out_type=[
        jax.ShapeDtypeStruct((NP, 2, OUT // 2), jnp.float32),
        jax.ShapeDtypeStruct((NP,), jnp.float32),
    ],
    mesh=plsc.VectorSubcoreMesh(
        core_axis_name="c", subcore_axis_name="s",
        num_cores=NC, num_subcores=NS),
    compiler_params=pltpu.CompilerParams(needs_layout_passes=False),
    scratch_types=[
        pltpu.VMEM((EPT,), jnp.int32),
        pltpu.VMEM((EPT,), jnp.int32),
        pltpu.VMEM((KCAP,), jnp.int32),
        pltpu.VMEM((DRC,), jnp.int32),
        pltpu.VMEM((GCH,), jnp.int32),
        pltpu.VMEM((GCH,), jnp.int32),
        pltpu.VMEM((GCH,), jnp.int32),
        pltpu.VMEM((GCH,), jnp.int32),
        pltpu.VMEM((GCH, 2, OUT // 2), jnp.float32),
        pltpu.VMEM((GCH, 2, OUT // 2), jnp.float32),
        pltpu.VMEM((DEGSZ,), jnp.float32),
        pltpu.VMEM_SHARED((QR, 2, OUT // 2), jnp.float32),
        pltpu.VMEM_SHARED((QR,), jnp.float32),
        pltpu.SemaphoreType.DMA,
    ],
  )(_sc_segsum_body)


def _segsum(xn, src, dst):
    agg, deg = _build_sc_segsum()(xn.reshape(NP, 2, OUT // 2), src, dst)
    return agg.reshape(NP, OUT), deg.reshape(NP, 1)


def _dot(a, b):
    return lax.dot_general(a, b, (((1,), (0,)), ((), ())),
                           preferred_element_type=jnp.float32)


def _row_mask(i):
    r = i * BM + lax.broadcasted_iota(jnp.int32, (BM, 1), 0)
    return r < N


def _full(shape):
    return pl.BlockSpec(shape, lambda i: (0,) * len(shape))


_ROW = pl.BlockSpec((BM, D), lambda i: (i, 0))
_DEGB = pl.BlockSpec((BM, 1), lambda i: (i, 0))


def _t1_body(h, ws0, wn0, b0, ws1, wn1, b1, ts0_o, tn0_o, ts1_o, tn1_o):
    x = h[...]
    ts0_o[...] = _dot(x, ws0[...]) + b0[...]
    tn0_o[...] = _dot(x, wn0[...])
    ts1_o[...] = _dot(x, ws1[...]) + b1[...]
    tn1_o[...] = _dot(x, wn1[...])


def _t1(h, ws0, wn0, b0, ws1, wn1, b1):
    return pl.pallas_call(
        _t1_body,
        grid=(NP // BM,),
        in_specs=[_ROW, _full((D, OUT)), _full((D, OUT)), _full((1, OUT)),
                  _full((D, OUT)), _full((D, OUT)), _full((1, OUT))],
        out_specs=[pl.BlockSpec((BM, OUT), lambda i: (i, 0))] * 4,
        out_shape=[jax.ShapeDtypeStruct((NP, OUT), jnp.float32)] * 4,
    )(h, ws0, wn0, b0.reshape(1, OUT), ws1, wn1, b1.reshape(1, OUT))


def _elu_norm(ts_ref, agg_ref, deg_ref):
    rdeg = 1.0 / jnp.maximum(deg_ref[...], 1.0)
    x = ts_ref[...] + agg_ref[...] * rdeg
    return jnp.where(x > 0, x, jnp.exp(jnp.minimum(x, 0.0)) - 1.0)


def _t2_body(ts_ref, agg_ref, deg_ref, ws, wn, b, ts_o, tn_o):
    e = _elu_norm(ts_ref, agg_ref, deg_ref)
    ts_o[...] = _dot(e, ws[...]) + b[...]
    tn_o[...] = jnp.where(_row_mask(pl.program_id(0)), _dot(e, wn[...]), 0.0)


_EIN = [_ROW, pl.BlockSpec((BM, OUT), lambda i: (i, 0)), _DEGB]


def _t2(ts, agg, deg, ws, wn, b):
    return pl.pallas_call(
        _t2_body,
        grid=(NP // BM,),
        in_specs=_EIN + [_full((D, OUT)), _full((D, OUT)), _full((1, OUT))],
        out_specs=[pl.BlockSpec((BM, OUT), lambda i: (i, 0))] * 2,
        out_shape=[jax.ShapeDtypeStruct((NP, OUT), jnp.float32)] * 2,
    )(ts, agg, deg, ws, wn, b.reshape(1, OUT))


def _t3_body(ts0, agg0, deg0, ts1, agg1, deg1, wa1, ba1, wa2, s_o):
    i = pl.program_id(0)

    @pl.when(i == 0)
    def _():
        s_o[...] = jnp.zeros((8, 128), jnp.float32)

    mask = _row_mask(i)
    w = []
    for ts_ref, agg_ref, deg_ref in ((ts0, agg0, deg0), (ts1, agg1, deg1)):
        e = _elu_norm(ts_ref, agg_ref, deg_ref)
        t = jnp.tanh(_dot(e, wa1[...]) + ba1[...])
        w.append(jnp.sum(jnp.where(mask, t * wa2[...], 0.0)))
    r = lax.broadcasted_iota(jnp.int32, (8, 128), 0)
    col = lax.broadcasted_iota(jnp.int32, (8, 128), 1)
    upd = jnp.where((r == 0) & (col == 0), w[0], 0.0) + \
          jnp.where((r == 0) & (col == 1), w[1], 0.0)
    s_o[...] += upd


def _t3(ts0, agg0, deg0, ts1, agg1, deg1, wa1, ba1, wa2):
    return pl.pallas_call(
        _t3_body,
        grid=(NP // BM,),
        in_specs=_EIN + _EIN +
                 [_full((OUT, HID)), _full((1, HID)), _full((1, HID))],
        out_specs=pl.BlockSpec((8, 128), lambda i: (0, 0)),
        out_shape=jax.ShapeDtypeStruct((8, 128), jnp.float32),
    )(ts0, agg0, deg0, ts1, agg1, deg1,
      wa1, ba1.reshape(1, HID), wa2.reshape(1, HID))


def _t4_body(ts0, agg0, deg0, ts1, agg1, deg1, s_ref, out_o):
    w0 = s_ref[0, 0] / N
    w1 = s_ref[0, 1] / N
    m = jnp.maximum(w0, w1)
    x0 = jnp.exp(w0 - m)
    x1 = jnp.exp(w1 - m)
    beta0 = x0 / (x0 + x1)
    e0 = _elu_norm(ts0, agg0, deg0)
    e1 = _elu_norm(ts1, agg1, deg1)
    out_o[...] = beta0 * e0 + (1.0 - beta0) * e1


def _t4(ts0, agg0, deg0, ts1, agg1, deg1, s):
    return pl.pallas_call(
        _t4_body,
        grid=(NP // BM,),
        in_specs=_EIN + _EIN + [_full((8, 128))],
        out_specs=pl.BlockSpec((BM, OUT), lambda i: (i, 0)),
        out_shape=jax.ShapeDtypeStruct((NP, OUT), jnp.float32),
    )(ts0, agg0, deg0, ts1, agg1, deg1, s)


def kernel(h, edge_index_0, edge_index_1,
           W_self_00, W_neigh_00, b_00, W_self_01, W_neigh_01, b_01,
           W_self_10, W_neigh_10, b_10, W_self_11, W_neigh_11, b_11,
           Wa1, ba1, Wa2):
    src0 = edge_index_0[0].astype(jnp.int32)
    dst0 = edge_index_0[1].astype(jnp.int32)
    src1 = edge_index_1[0].astype(jnp.int32)
    dst1 = edge_index_1[1].astype(jnp.int32)

    hp = jnp.zeros((NP, D), jnp.float32).at[:N].set(h.astype(jnp.float32))

    ts00, tn00, ts10, tn10 = _t1(hp, W_self_00, W_neigh_00, b_00,
                                 W_self_10, W_neigh_10, b_10)
    agg00, deg0 = _segsum(tn00, src0, dst0)
    agg10, deg1 = _segsum(tn10, src1, dst1)

    ts01, tn01 = _t2(ts00, agg00, deg0, W_self_01, W_neigh_01, b_01)
    agg01, _ = _segsum(tn01, src0, dst0)
    ts11, tn11 = _t2(ts10, agg10, deg1, W_self_11, W_neigh_11, b_11)
    agg11, _ = _segsum(tn11, src1, dst1)

    s = _t3(ts01, agg01, deg0, ts11, agg11, deg1, Wa1, ba1, Wa2)
    out = _t4(ts01, agg01, deg0, ts11, agg11, deg1, s)
    return out[:N]


# speedup vs baseline: 1.3334x; 1.2036x over previous
import functools

import jax
import jax.numpy as jnp
from jax import lax
from jax.experimental import pallas as pl
from jax.experimental.pallas import tpu as pltpu
from jax.experimental.pallas import tpu_sc as plsc

N = 10000
NP = 10240
D = 256
OUT = 256
HID = 128
E = 160000

NC = 2
NS = 16
NQ = 4
QR = NP // NQ
RPT = QR // NS
EPT = E // NS
GCH = 128
PADROW = NP - 8
KCAP = EPT + GCH + 64
TRASH = KCAP - 16

BM = 512


def _sc_segsum_body(xn, srcl, dstl, agg_o, deg_o,
                    src_buf, dst_buf, ksrc, kdst, sidx, didx, rows,
                    degt, zbuf, acc, sdeg, sem):
    c = lax.axis_index("c")
    s = lax.axis_index("s")
    zero16f = jnp.zeros((16,), jnp.float32)
    iota16 = lax.iota(jnp.int32, 16)

    def _zb(i, _):
        def _zl(k, _):
            zbuf[i, 0, pl.ds(k * 16, 16)] = zero16f
            zbuf[i, 1, pl.ds(k * 16, 16)] = zero16f
            return 0
        return lax.fori_loop(0, 8, _zl, 0)
    lax.fori_loop(0, 16, _zb, 0)

    ebase = s * EPT
    pltpu.sync_copy(srcl.at[pl.ds(ebase, EPT)], src_buf)
    pltpu.sync_copy(dstl.at[pl.ds(ebase, EPT)], dst_buf)

    base = s * RPT

    for p in range(2):
        q = 2 * p + c
        lo = q * QR

        def _zd(i, _):
            degt[pl.ds(i * 16, 16)] = zero16f
            return 0
        lax.fori_loop(0, QR // 16, _zd, 0)
        for off in range(0, RPT, 16):
            pltpu.sync_copy(zbuf, acc.at[pl.ds(base + off, 16)])

        @pl.when(s == 0)
        def _():
            pltpu.sync_copy(degt, sdeg)

        plsc.subcore_barrier()

        def _filt(j, cnt):
            sv = src_buf[pl.ds(j * 16, 16)]
            dv = dst_buf[pl.ds(j * 16, 16)]
            dl = dv - lo
            m = (dl >= 0) & (dl < QR)
            mi = jnp.where(m, 1, 0).astype(jnp.int32)
            inc = plsc.cumsum(mi)
            widx = jnp.where(m, cnt + inc - mi, TRASH + iota16)
            plsc.store_scatter(ksrc, [widx], sv)
            plsc.store_scatter(kdst, [widx], dl)
            return cnt + jnp.sum(mi)
        cnt = lax.fori_loop(0, EPT // 16, _filt, jnp.int32(0))

        pad16 = jnp.full((16,), PADROW, jnp.int32)
        zero16i = jnp.zeros((16,), jnp.int32)
        for k in range(GCH // 16):
            ksrc[pl.ds(cnt + k * 16, 16)] = pad16
            kdst[pl.ds(cnt + k * 16, 16)] = zero16i
        ngc = (cnt + GCH - 1) // GCH

        ones16 = jnp.ones((16,), jnp.float32)

        def _gat(g, _):
            gb = g * GCH
            for k in range(GCH // 16):
                sidx[pl.ds(k * 16, 16)] = ksrc[pl.ds(gb + k * 16, 16)]
                didx[pl.ds(k * 16, 16)] = kdst[pl.ds(gb + k * 16, 16)]
            pltpu.async_copy(xn.at[sidx], rows, sem).wait()
            pltpu.sync_copy(rows, acc.at[didx], add=True)
            for k in range(GCH // 16):
                dv = didx[pl.ds(k * 16, 16)]
                ones = jnp.where(gb + k * 16 + iota16 < cnt, 1.0, 0.0)
                plsc.addupdate_scatter(degt, [dv], ones)
            return 0
        lax.fori_loop(0, ngc, _gat, 0)

        def _dr(g, _):
            gb = g * GCH
            for k in range(GCH // 16):
                didx[pl.ds(k * 16, 16)] = iota16 + (gb + k * 16)
            pltpu.sync_copy(degt.at[pl.ds(gb, GCH)], sdeg.at[didx], add=True)
            return 0
        lax.fori_loop(0, QR // GCH, _dr, 0)

        plsc.subcore_barrier()

        pltpu.sync_copy(acc.at[pl.ds(base, RPT)],
                        agg_o.at[pl.ds(lo + base, RPT)])

        @pl.when(s == 0)
        def _():
            pltpu.sync_copy(sdeg, deg_o.at[pl.ds(lo, QR)])

        plsc.subcore_barrier()


@functools.lru_cache(maxsize=1)
def _build_sc_segsum():
  return functools.partial(
    pl.kernel,
    out_type=[
        jax.ShapeDtypeStruct((NP, 2, OUT // 2), jnp.float32),
        jax.ShapeDtypeStruct((NP,), jnp.float32),
    ],
    mesh=plsc.VectorSubcoreMesh(
        core_axis_name="c", subcore_axis_name="s",
        num_cores=NC, num_subcores=NS),
    compiler_params=pltpu.CompilerParams(needs_layout_passes=False),
    scratch_types=[
        pltpu.VMEM((EPT,), jnp.int32),
        pltpu.VMEM((EPT,), jnp.int32),
        pltpu.VMEM((KCAP,), jnp.int32),
        pltpu.VMEM((KCAP,), jnp.int32),
        pltpu.VMEM((GCH,), jnp.int32),
        pltpu.VMEM((GCH,), jnp.int32),
        pltpu.VMEM((GCH, 2, OUT // 2), jnp.float32),
        pltpu.VMEM((QR,), jnp.float32),
        pltpu.VMEM((16, 2, OUT // 2), jnp.float32),
        pltpu.VMEM_SHARED((QR, 2, OUT // 2), jnp.float32),
        pltpu.VMEM_SHARED((QR,), jnp.float32),
        pltpu.SemaphoreType.DMA,
    ],
  )(_sc_segsum_body)


def _segsum(xn, src, dst):
    agg, deg = _build_sc_segsum()(xn.reshape(NP, 2, OUT // 2), src, dst)
    return agg.reshape(NP, OUT), deg.reshape(NP, 1)


def _dot(a, b):
    return lax.dot_general(a, b, (((1,), (0,)), ((), ())),
                           preferred_element_type=jnp.float32)


def _row_mask(i):
    r = i * BM + lax.broadcasted_iota(jnp.int32, (BM, 1), 0)
    return r < N


def _full(shape):
    return pl.BlockSpec(shape, lambda i: (0,) * len(shape))


_ROW = pl.BlockSpec((BM, D), lambda i: (i, 0))
_DEGB = pl.BlockSpec((BM, 1), lambda i: (i, 0))


def _t1_body(h, ws0, wn0, b0, ws1, wn1, b1, ts0_o, tn0_o, ts1_o, tn1_o):
    x = h[...]
    ts0_o[...] = _dot(x, ws0[...]) + b0[...]
    tn0_o[...] = _dot(x, wn0[...])
    ts1_o[...] = _dot(x, ws1[...]) + b1[...]
    tn1_o[...] = _dot(x, wn1[...])


def _t1(h, ws0, wn0, b0, ws1, wn1, b1):
    return pl.pallas_call(
        _t1_body,
        grid=(NP // BM,),
        in_specs=[_ROW, _full((D, OUT)), _full((D, OUT)), _full((1, OUT)),
                  _full((D, OUT)), _full((D, OUT)), _full((1, OUT))],
        out_specs=[pl.BlockSpec((BM, OUT), lambda i: (i, 0))] * 4,
        out_shape=[jax.ShapeDtypeStruct((NP, OUT), jnp.float32)] * 4,
    )(h, ws0, wn0, b0.reshape(1, OUT), ws1, wn1, b1.reshape(1, OUT))


def _elu_norm(ts_ref, agg_ref, deg_ref):
    rdeg = 1.0 / jnp.maximum(deg_ref[...], 1.0)
    x = ts_ref[...] + agg_ref[...] * rdeg
    return jnp.where(x > 0, x, jnp.exp(jnp.minimum(x, 0.0)) - 1.0)


def _t2_body(ts_ref, agg_ref, deg_ref, ws, wn, b, ts_o, tn_o):
    e = _elu_norm(ts_ref, agg_ref, deg_ref)
    ts_o[...] = _dot(e, ws[...]) + b[...]
    tn_o[...] = jnp.where(_row_mask(pl.program_id(0)), _dot(e, wn[...]), 0.0)


_EIN = [_ROW, pl.BlockSpec((BM, OUT), lambda i: (i, 0)), _DEGB]


def _t2(ts, agg, deg, ws, wn, b):
    return pl.pallas_call(
        _t2_body,
        grid=(NP // BM,),
        in_specs=_EIN + [_full((D, OUT)), _full((D, OUT)), _full((1, OUT))],
        out_specs=[pl.BlockSpec((BM, OUT), lambda i: (i, 0))] * 2,
        out_shape=[jax.ShapeDtypeStruct((NP, OUT), jnp.float32)] * 2,
    )(ts, agg, deg, ws, wn, b.reshape(1, OUT))


def _t3_body(ts0, agg0, deg0, ts1, agg1, deg1, wa1, ba1, wa2, s_o):
    i = pl.program_id(0)

    @pl.when(i == 0)
    def _():
        s_o[...] = jnp.zeros((8, 128), jnp.float32)

    mask = _row_mask(i)
    w = []
    for ts_ref, agg_ref, deg_ref in ((ts0, agg0, deg0), (ts1, agg1, deg1)):
        e = _elu_norm(ts_ref, agg_ref, deg_ref)
        t = jnp.tanh(_dot(e, wa1[...]) + ba1[...])
        w.append(jnp.sum(jnp.where(mask, t * wa2[...], 0.0)))
    r = lax.broadcasted_iota(jnp.int32, (8, 128), 0)
    col = lax.broadcasted_iota(jnp.int32, (8, 128), 1)
    upd = jnp.where((r == 0) & (col == 0), w[0], 0.0) + \
          jnp.where((r == 0) & (col == 1), w[1], 0.0)
    s_o[...] += upd


def _t3(ts0, agg0, deg0, ts1, agg1, deg1, wa1, ba1, wa2):
    return pl.pallas_call(
        _t3_body,
        grid=(NP // BM,),
        in_specs=_EIN + _EIN +
                 [_full((OUT, HID)), _full((1, HID)), _full((1, HID))],
        out_specs=pl.BlockSpec((8, 128), lambda i: (0, 0)),
        out_shape=jax.ShapeDtypeStruct((8, 128), jnp.float32),
    )(ts0, agg0, deg0, ts1, agg1, deg1,
      wa1, ba1.reshape(1, HID), wa2.reshape(1, HID))


def _t4_body(ts0, agg0, deg0, ts1, agg1, deg1, s_ref, out_o):
    w0 = s_ref[0, 0] / N
    w1 = s_ref[0, 1] / N
    m = jnp.maximum(w0, w1)
    x0 = jnp.exp(w0 - m)
    x1 = jnp.exp(w1 - m)
    beta0 = x0 / (x0 + x1)
    e0 = _elu_norm(ts0, agg0, deg0)
    e1 = _elu_norm(ts1, agg1, deg1)
    out_o[...] = beta0 * e0 + (1.0 - beta0) * e1


def _t4(ts0, agg0, deg0, ts1, agg1, deg1, s):
    return pl.pallas_call(
        _t4_body,
        grid=(NP // BM,),
        in_specs=_EIN + _EIN + [_full((8, 128))],
        out_specs=pl.BlockSpec((BM, OUT), lambda i: (i, 0)),
        out_shape=jax.ShapeDtypeStruct((NP, OUT), jnp.float32),
    )(ts0, agg0, deg0, ts1, agg1, deg1, s)


def kernel(h, edge_index_0, edge_index_1,
           W_self_00, W_neigh_00, b_00, W_self_01, W_neigh_01, b_01,
           W_self_10, W_neigh_10, b_10, W_self_11, W_neigh_11, b_11,
           Wa1, ba1, Wa2):
    src0 = edge_index_0[0].astype(jnp.int32)
    dst0 = edge_index_0[1].astype(jnp.int32)
    src1 = edge_index_1[0].astype(jnp.int32)
    dst1 = edge_index_1[1].astype(jnp.int32)

    hp = jnp.zeros((NP, D), jnp.float32).at[:N].set(h.astype(jnp.float32))

    ts00, tn00, ts10, tn10 = _t1(hp, W_self_00, W_neigh_00, b_00,
                                 W_self_10, W_neigh_10, b_10)
    agg00, deg0 = _segsum(tn00, src0, dst0)
    agg10, deg1 = _segsum(tn10, src1, dst1)

    ts01, tn01 = _t2(ts00, agg00, deg0, W_self_01, W_neigh_01, b_01)
    agg01, _ = _segsum(tn01, src0, dst0)
    ts11, tn11 = _t2(ts10, agg10, deg1, W_self_11, W_neigh_11, b_11)
    agg11, _ = _segsum(tn11, src1, dst1)

    s = _t3(ts01, agg01, deg0, ts11, agg11, deg1, Wa1, ba1, Wa2)
    out = _t4(ts01, agg01, deg0, ts11, agg11, deg1, s)
    return out[:N]


# speedup vs baseline: 1.3360x; 1.0019x over previous
import functools

import jax
import jax.numpy as jnp
from jax import lax
from jax.experimental import pallas as pl
from jax.experimental.pallas import tpu as pltpu
from jax.experimental.pallas import tpu_sc as plsc

N = 10000
NP = 10240
D = 256
OUT = 256
HID = 128
E = 160000

NC = 2
NS = 16
NQ = 4
QR = NP // NQ
RPT = QR // NS
EPT = E // NS
GCH = 128
PADROW = NP - 8
KCAP = EPT + GCH + 64
TRASH = KCAP - 16

BM = 512


def _sc_segsum_body(xn, srcl, dstl, agg_o, deg_o,
                    src_buf, dst_buf, ksrc, kdst, sidx, didx, rows,
                    degt, zbuf, acc, sdeg, sem):
    c = lax.axis_index("c")
    s = lax.axis_index("s")
    zero16f = jnp.zeros((16,), jnp.float32)
    iota16 = lax.iota(jnp.int32, 16)

    def _zb(i, _):
        def _zl(k, _):
            zbuf[i, 0, pl.ds(k * 16, 16)] = zero16f
            zbuf[i, 1, pl.ds(k * 16, 16)] = zero16f
            return 0
        return lax.fori_loop(0, 8, _zl, 0)
    lax.fori_loop(0, 16, _zb, 0)

    ebase = s * EPT
    pltpu.sync_copy(srcl.at[pl.ds(ebase, EPT)], src_buf)
    pltpu.sync_copy(dstl.at[pl.ds(ebase, EPT)], dst_buf)

    base = s * RPT

    for p in range(2):
        q = 2 * p + c
        lo = q * QR

        def _zd(i, _):
            degt[pl.ds(i * 16, 16)] = zero16f
            return 0
        lax.fori_loop(0, QR // 16, _zd, 0)
        for off in range(0, RPT, 16):
            pltpu.sync_copy(zbuf, acc.at[pl.ds(base + off, 16)])

        @pl.when(s == 0)
        def _():
            pltpu.sync_copy(degt, sdeg)

        plsc.subcore_barrier()

        def _filt(j, cnt):
            sv = src_buf[pl.ds(j * 16, 16)]
            dv = dst_buf[pl.ds(j * 16, 16)]
            dl = dv - lo
            m = (dl >= 0) & (dl < QR)
            mi = jnp.where(m, 1, 0).astype(jnp.int32)
            inc = plsc.cumsum(mi)
            widx = jnp.where(m, cnt + inc - mi, TRASH + iota16)
            plsc.store_scatter(ksrc, [widx], sv)
            plsc.store_scatter(kdst, [widx], dl)
            return cnt + inc[15]
        cnt = lax.fori_loop(0, EPT // 16, _filt, jnp.int32(0))

        pad16 = jnp.full((16,), PADROW, jnp.int32)
        zero16i = jnp.zeros((16,), jnp.int32)
        for k in range(GCH // 16):
            ksrc[pl.ds(cnt + k * 16, 16)] = pad16
            kdst[pl.ds(cnt + k * 16, 16)] = zero16i
        ngc = (cnt + GCH - 1) // GCH

        ones16 = jnp.ones((16,), jnp.float32)

        def _gat(g, last):
            gb = g * GCH
            for k in range(GCH // 16):
                didx[pl.ds(k * 16, 16)] = kdst[pl.ds(gb + k * 16, 16)]
            pltpu.async_copy(
                xn.at[ksrc.at[pl.ds(gb, GCH)]], rows, sem).wait()
            pltpu.sync_copy(rows, acc.at[didx], add=True)
            for k in range(GCH // 16):
                dv = didx[pl.ds(k * 16, 16)]
                if last:
                    ones = jnp.where(gb + k * 16 + iota16 < cnt, 1.0, 0.0)
                else:
                    ones = ones16
                plsc.addupdate_scatter(degt, [dv], ones)

        def _gat_full(g, _):
            _gat(g, False)
            return 0
        lax.fori_loop(0, ngc - 1, _gat_full, 0)

        @pl.when(ngc > 0)
        def _():
            _gat(ngc - 1, True)

        def _dr(g, _):
            gb = g * GCH
            for k in range(GCH // 16):
                didx[pl.ds(k * 16, 16)] = iota16 + (gb + k * 16)
            pltpu.sync_copy(degt.at[pl.ds(gb, GCH)], sdeg.at[didx], add=True)
            return 0
        lax.fori_loop(0, QR // GCH, _dr, 0)

        plsc.subcore_barrier()

        pltpu.sync_copy(acc.at[pl.ds(base, RPT)],
                        agg_o.at[pl.ds(lo + base, RPT)])

        @pl.when(s == 0)
        def _():
            pltpu.sync_copy(sdeg, deg_o.at[pl.ds(lo, QR)])

        plsc.subcore_barrier()


@functools.lru_cache(maxsize=1)
def _build_sc_segsum():
  return functools.partial(
    pl.kernel,
    out_type=[
        jax.ShapeDtypeStruct((NP, 2, OUT // 2), jnp.float32),
        jax.ShapeDtypeStruct((NP,), jnp.float32),
    ],
    mesh=plsc.VectorSubcoreMesh(
        core_axis_name="c", subcore_axis_name="s",
        num_cores=NC, num_subcores=NS),
    compiler_params=pltpu.CompilerParams(needs_layout_passes=False),
    scratch_types=[
        pltpu.VMEM((EPT,), jnp.int32),
        pltpu.VMEM((EPT,), jnp.int32),
        pltpu.VMEM((KCAP,), jnp.int32),
        pltpu.VMEM((KCAP,), jnp.int32),
        pltpu.VMEM((GCH,), jnp.int32),
        pltpu.VMEM((GCH,), jnp.int32),
        pltpu.VMEM((GCH, 2, OUT // 2), jnp.float32),
        pltpu.VMEM((QR,), jnp.float32),
        pltpu.VMEM((16, 2, OUT // 2), jnp.float32),
        pltpu.VMEM_SHARED((QR, 2, OUT // 2), jnp.float32),
        pltpu.VMEM_SHARED((QR,), jnp.float32),
        pltpu.SemaphoreType.DMA,
    ],
  )(_sc_segsum_body)


def _segsum(xn, src, dst):
    agg, deg = _build_sc_segsum()(xn.reshape(NP, 2, OUT // 2), src, dst)
    return agg.reshape(NP, OUT), deg.reshape(NP, 1)


def _dot(a, b):
    return lax.dot_general(a, b, (((1,), (0,)), ((), ())),
                           preferred_element_type=jnp.float32)


def _row_mask(i):
    r = i * BM + lax.broadcasted_iota(jnp.int32, (BM, 1), 0)
    return r < N


def _full(shape):
    return pl.BlockSpec(shape, lambda i: (0,) * len(shape))


_ROW = pl.BlockSpec((BM, D), lambda i: (i, 0))
_DEGB = pl.BlockSpec((BM, 1), lambda i: (i, 0))


def _t1_body(h, ws0, wn0, b0, ws1, wn1, b1, ts0_o, tn0_o, ts1_o, tn1_o):
    x = h[...]
    ts0_o[...] = _dot(x, ws0[...]) + b0[...]
    tn0_o[...] = _dot(x, wn0[...])
    ts1_o[...] = _dot(x, ws1[...]) + b1[...]
    tn1_o[...] = _dot(x, wn1[...])


def _t1(h, ws0, wn0, b0, ws1, wn1, b1):
    return pl.pallas_call(
        _t1_body,
        grid=(NP // BM,),
        in_specs=[_ROW, _full((D, OUT)), _full((D, OUT)), _full((1, OUT)),
                  _full((D, OUT)), _full((D, OUT)), _full((1, OUT))],
        out_specs=[pl.BlockSpec((BM, OUT), lambda i: (i, 0))] * 4,
        out_shape=[jax.ShapeDtypeStruct((NP, OUT), jnp.float32)] * 4,
    )(h, ws0, wn0, b0.reshape(1, OUT), ws1, wn1, b1.reshape(1, OUT))


def _elu_norm(ts_ref, agg_ref, deg_ref):
    rdeg = 1.0 / jnp.maximum(deg_ref[...], 1.0)
    x = ts_ref[...] + agg_ref[...] * rdeg
    return jnp.where(x > 0, x, jnp.exp(jnp.minimum(x, 0.0)) - 1.0)


def _t2_body(ts_ref, agg_ref, deg_ref, ws, wn, b, ts_o, tn_o):
    e = _elu_norm(ts_ref, agg_ref, deg_ref)
    ts_o[...] = _dot(e, ws[...]) + b[...]
    tn_o[...] = jnp.where(_row_mask(pl.program_id(0)), _dot(e, wn[...]), 0.0)


_EIN = [_ROW, pl.BlockSpec((BM, OUT), lambda i: (i, 0)), _DEGB]


def _t2(ts, agg, deg, ws, wn, b):
    return pl.pallas_call(
        _t2_body,
        grid=(NP // BM,),
        in_specs=_EIN + [_full((D, OUT)), _full((D, OUT)), _full((1, OUT))],
        out_specs=[pl.BlockSpec((BM, OUT), lambda i: (i, 0))] * 2,
        out_shape=[jax.ShapeDtypeStruct((NP, OUT), jnp.float32)] * 2,
    )(ts, agg, deg, ws, wn, b.reshape(1, OUT))


def _t3_body(ts0, agg0, deg0, ts1, agg1, deg1, wa1, ba1, wa2, s_o):
    i = pl.program_id(0)

    @pl.when(i == 0)
    def _():
        s_o[...] = jnp.zeros((8, 128), jnp.float32)

    mask = _row_mask(i)
    w = []
    for ts_ref, agg_ref, deg_ref in ((ts0, agg0, deg0), (ts1, agg1, deg1)):
        e = _elu_norm(ts_ref, agg_ref, deg_ref)
        t = jnp.tanh(_dot(e, wa1[...]) + ba1[...])
        w.append(jnp.sum(jnp.where(mask, t * wa2[...], 0.0)))
    r = lax.broadcasted_iota(jnp.int32, (8, 128), 0)
    col = lax.broadcasted_iota(jnp.int32, (8, 128), 1)
    upd = jnp.where((r == 0) & (col == 0), w[0], 0.0) + \
          jnp.where((r == 0) & (col == 1), w[1], 0.0)
    s_o[...] += upd


def _t3(ts0, agg0, deg0, ts1, agg1, deg1, wa1, ba1, wa2):
    return pl.pallas_call(
        _t3_body,
        grid=(NP // BM,),
        in_specs=_EIN + _EIN +
                 [_full((OUT, HID)), _full((1, HID)), _full((1, HID))],
        out_specs=pl.BlockSpec((8, 128), lambda i: (0, 0)),
        out_shape=jax.ShapeDtypeStruct((8, 128), jnp.float32),
    )(ts0, agg0, deg0, ts1, agg1, deg1,
      wa1, ba1.reshape(1, HID), wa2.reshape(1, HID))


def _t4_body(ts0, agg0, deg0, ts1, agg1, deg1, s_ref, out_o):
    w0 = s_ref[0, 0] / N
    w1 = s_ref[0, 1] / N
    m = jnp.maximum(w0, w1)
    x0 = jnp.exp(w0 - m)
    x1 = jnp.exp(w1 - m)
    beta0 = x0 / (x0 + x1)
    e0 = _elu_norm(ts0, agg0, deg0)
    e1 = _elu_norm(ts1, agg1, deg1)
    out_o[...] = beta0 * e0 + (1.0 - beta0) * e1


def _t4(ts0, agg0, deg0, ts1, agg1, deg1, s):
    return pl.pallas_call(
        _t4_body,
        grid=(NP // BM,),
        in_specs=_EIN + _EIN + [_full((8, 128))],
        out_specs=pl.BlockSpec((BM, OUT), lambda i: (i, 0)),
        out_shape=jax.ShapeDtypeStruct((NP, OUT), jnp.float32),
    )(ts0, agg0, deg0, ts1, agg1, deg1, s)


def kernel(h, edge_index_0, edge_index_1,
           W_self_00, W_neigh_00, b_00, W_self_01, W_neigh_01, b_01,
           W_self_10, W_neigh_10, b_10, W_self_11, W_neigh_11, b_11,
           Wa1, ba1, Wa2):
    src0 = edge_index_0[0].astype(jnp.int32)
    dst0 = edge_index_0[1].astype(jnp.int32)
    src1 = edge_index_1[0].astype(jnp.int32)
    dst1 = edge_index_1[1].astype(jnp.int32)

    hp = jnp.zeros((NP, D), jnp.float32).at[:N].set(h.astype(jnp.float32))

    ts00, tn00, ts10, tn10 = _t1(hp, W_self_00, W_neigh_00, b_00,
                                 W_self_10, W_neigh_10, b_10)
    agg00, deg0 = _segsum(tn00, src0, dst0)
    agg10, deg1 = _segsum(tn10, src1, dst1)

    ts01, tn01 = _t2(ts00, agg00, deg0, W_self_01, W_neigh_01, b_01)
    agg01, _ = _segsum(tn01, src0, dst0)
    ts11, tn11 = _t2(ts10, agg10, deg1, W_self_11, W_neigh_11, b_11)
    agg11, _ = _segsum(tn11, src1, dst1)

    s = _t3(ts01, agg01, deg0, ts11, agg11, deg1, Wa1, ba1, Wa2)
    out = _t4(ts01, agg01, deg0, ts11, agg11, deg1, s)
    return out[:N]


# speedup vs baseline: 1.3626x; 1.0199x over previous
import functools

import jax
import jax.numpy as jnp
from jax import lax
from jax.experimental import pallas as pl
from jax.experimental.pallas import tpu as pltpu
from jax.experimental.pallas import tpu_sc as plsc

N = 10000
NP = 10240
D = 256
OUT = 256
HID = 128
E = 160000

NC = 2
NS = 16
NQ = 4
QR = NP // NQ
RPT = QR // NS
EPT = E // NS
GCH = 128
PADROW = NP - 8
KCAP = EPT + GCH + 64
TRASH = KCAP - 16

BM = 2048


def _sc_segsum_body(xn, srcl, dstl, agg_o, deg_o,
                    src_buf, dst_buf, ksrc, kdst, sidx, didx, rows,
                    degt, zbuf, acc, sdeg, sem):
    c = lax.axis_index("c")
    s = lax.axis_index("s")
    zero16f = jnp.zeros((16,), jnp.float32)
    iota16 = lax.iota(jnp.int32, 16)

    def _zb(i, _):
        def _zl(k, _):
            zbuf[i, 0, pl.ds(k * 16, 16)] = zero16f
            zbuf[i, 1, pl.ds(k * 16, 16)] = zero16f
            return 0
        return lax.fori_loop(0, 8, _zl, 0)
    lax.fori_loop(0, 16, _zb, 0)

    ebase = s * EPT
    pltpu.sync_copy(srcl.at[pl.ds(ebase, EPT)], src_buf)
    pltpu.sync_copy(dstl.at[pl.ds(ebase, EPT)], dst_buf)

    base = s * RPT

    for p in range(2):
        q = 2 * p + c
        lo = q * QR

        def _zd(i, _):
            degt[pl.ds(i * 16, 16)] = zero16f
            return 0
        lax.fori_loop(0, QR // 16, _zd, 0)
        for off in range(0, RPT, 16):
            pltpu.sync_copy(zbuf, acc.at[pl.ds(base + off, 16)])

        @pl.when(s == 0)
        def _():
            pltpu.sync_copy(degt, sdeg)

        plsc.subcore_barrier()

        def _filt(j, cnt):
            sv = src_buf[pl.ds(j * 16, 16)]
            dv = dst_buf[pl.ds(j * 16, 16)]
            dl = dv - lo
            m = (dl >= 0) & (dl < QR)
            mi = jnp.where(m, 1, 0).astype(jnp.int32)
            inc = plsc.cumsum(mi)
            widx = jnp.where(m, cnt + inc - mi, TRASH + iota16)
            plsc.store_scatter(ksrc, [widx], sv)
            plsc.store_scatter(kdst, [widx], dl)
            return cnt + inc[15]
        cnt = lax.fori_loop(0, EPT // 16, _filt, jnp.int32(0))

        pad16 = jnp.full((16,), PADROW, jnp.int32)
        zero16i = jnp.zeros((16,), jnp.int32)
        for k in range(GCH // 16):
            ksrc[pl.ds(cnt + k * 16, 16)] = pad16
            kdst[pl.ds(cnt + k * 16, 16)] = zero16i
        ngc = (cnt + GCH - 1) // GCH

        ones16 = jnp.ones((16,), jnp.float32)

        def _gat(g, last):
            gb = g * GCH
            for k in range(GCH // 16):
                didx[pl.ds(k * 16, 16)] = kdst[pl.ds(gb + k * 16, 16)]
            pltpu.async_copy(
                xn.at[ksrc.at[pl.ds(gb, GCH)]], rows, sem).wait()
            pltpu.sync_copy(rows, acc.at[didx], add=True)
            for k in range(GCH // 16):
                dv = didx[pl.ds(k * 16, 16)]
                if last:
                    ones = jnp.where(gb + k * 16 + iota16 < cnt, 1.0, 0.0)
                else:
                    ones = ones16
                plsc.addupdate_scatter(degt, [dv], ones)

        def _gat_full(g, _):
            _gat(g, False)
            return 0
        lax.fori_loop(0, ngc - 1, _gat_full, 0)

        @pl.when(ngc > 0)
        def _():
            _gat(ngc - 1, True)

        def _dr(g, _):
            gb = g * GCH
            for k in range(GCH // 16):
                didx[pl.ds(k * 16, 16)] = iota16 + (gb + k * 16)
            pltpu.sync_copy(degt.at[pl.ds(gb, GCH)], sdeg.at[didx], add=True)
            return 0
        lax.fori_loop(0, QR // GCH, _dr, 0)

        plsc.subcore_barrier()

        pltpu.sync_copy(acc.at[pl.ds(base, RPT)],
                        agg_o.at[pl.ds(lo + base, RPT)])

        @pl.when(s == 0)
        def _():
            pltpu.sync_copy(sdeg, deg_o.at[pl.ds(lo, QR)])

        plsc.subcore_barrier()


@functools.lru_cache(maxsize=1)
def _build_sc_segsum():
  return functools.partial(
    pl.kernel,
    out_type=[
        jax.ShapeDtypeStruct((NP, 2, OUT // 2), jnp.float32),
        jax.ShapeDtypeStruct((NP,), jnp.float32),
    ],
    mesh=plsc.VectorSubcoreMesh(
        core_axis_name="c", subcore_axis_name="s",
        num_cores=NC, num_subcores=NS),
    compiler_params=pltpu.CompilerParams(needs_layout_passes=False),
    scratch_types=[
        pltpu.VMEM((EPT,), jnp.int32),
        pltpu.VMEM((EPT,), jnp.int32),
        pltpu.VMEM((KCAP,), jnp.int32),
        pltpu.VMEM((KCAP,), jnp.int32),
        pltpu.VMEM((GCH,), jnp.int32),
        pltpu.VMEM((GCH,), jnp.int32),
        pltpu.VMEM((GCH, 2, OUT // 2), jnp.float32),
        pltpu.VMEM((QR,), jnp.float32),
        pltpu.VMEM((16, 2, OUT // 2), jnp.float32),
        pltpu.VMEM_SHARED((QR, 2, OUT // 2), jnp.float32),
        pltpu.VMEM_SHARED((QR,), jnp.float32),
        pltpu.SemaphoreType.DMA,
    ],
  )(_sc_segsum_body)


def _segsum(xn, src, dst):
    agg, deg = _build_sc_segsum()(xn.reshape(NP, 2, OUT // 2), src, dst)
    return agg.reshape(NP, OUT), deg.reshape(NP, 1)


def _dot(a, b):
    return lax.dot_general(a, b, (((1,), (0,)), ((), ())),
                           preferred_element_type=jnp.float32)


def _row_mask(i):
    r = i * BM + lax.broadcasted_iota(jnp.int32, (BM, 1), 0)
    return r < N


def _full(shape):
    return pl.BlockSpec(shape, lambda i: (0,) * len(shape))


_ROW = pl.BlockSpec((BM, D), lambda i: (i, 0))
_DEGB = pl.BlockSpec((BM, 1), lambda i: (i, 0))


def _t1_body(h, ws0, wn0, b0, ws1, wn1, b1, ts0_o, tn0_o, ts1_o, tn1_o):
    x = h[...]
    ts0_o[...] = _dot(x, ws0[...]) + b0[...]
    tn0_o[...] = _dot(x, wn0[...])
    ts1_o[...] = _dot(x, ws1[...]) + b1[...]
    tn1_o[...] = _dot(x, wn1[...])


def _t1(h, ws0, wn0, b0, ws1, wn1, b1):
    return pl.pallas_call(
        _t1_body,
        grid=(NP // BM,),
        in_specs=[_ROW, _full((D, OUT)), _full((D, OUT)), _full((1, OUT)),
                  _full((D, OUT)), _full((D, OUT)), _full((1, OUT))],
        out_specs=[pl.BlockSpec((BM, OUT), lambda i: (i, 0))] * 4,
        out_shape=[jax.ShapeDtypeStruct((NP, OUT), jnp.float32)] * 4,
    )(h, ws0, wn0, b0.reshape(1, OUT), ws1, wn1, b1.reshape(1, OUT))


def _elu_norm(ts_ref, agg_ref, deg_ref):
    rdeg = 1.0 / jnp.maximum(deg_ref[...], 1.0)
    x = ts_ref[...] + agg_ref[...] * rdeg
    return jnp.where(x > 0, x, jnp.exp(jnp.minimum(x, 0.0)) - 1.0)


def _t2_body(ts_ref, agg_ref, deg_ref, ws, wn, b, ts_o, tn_o):
    e = _elu_norm(ts_ref, agg_ref, deg_ref)
    ts_o[...] = _dot(e, ws[...]) + b[...]
    tn_o[...] = jnp.where(_row_mask(pl.program_id(0)), _dot(e, wn[...]), 0.0)


_EIN = [_ROW, pl.BlockSpec((BM, OUT), lambda i: (i, 0)), _DEGB]


def _t2(ts, agg, deg, ws, wn, b):
    return pl.pallas_call(
        _t2_body,
        grid=(NP // BM,),
        in_specs=_EIN + [_full((D, OUT)), _full((D, OUT)), _full((1, OUT))],
        out_specs=[pl.BlockSpec((BM, OUT), lambda i: (i, 0))] * 2,
        out_shape=[jax.ShapeDtypeStruct((NP, OUT), jnp.float32)] * 2,
    )(ts, agg, deg, ws, wn, b.reshape(1, OUT))


def _t3_body(ts0, agg0, deg0, ts1, agg1, deg1, wa1, ba1, wa2, s_o):
    i = pl.program_id(0)

    @pl.when(i == 0)
    def _():
        s_o[...] = jnp.zeros((8, 128), jnp.float32)

    mask = _row_mask(i)
    w = []
    for ts_ref, agg_ref, deg_ref in ((ts0, agg0, deg0), (ts1, agg1, deg1)):
        e = _elu_norm(ts_ref, agg_ref, deg_ref)
        t = jnp.tanh(_dot(e, wa1[...]) + ba1[...])
        w.append(jnp.sum(jnp.where(mask, t * wa2[...], 0.0)))
    r = lax.broadcasted_iota(jnp.int32, (8, 128), 0)
    col = lax.broadcasted_iota(jnp.int32, (8, 128), 1)
    upd = jnp.where((r == 0) & (col == 0), w[0], 0.0) + \
          jnp.where((r == 0) & (col == 1), w[1], 0.0)
    s_o[...] += upd


def _t3(ts0, agg0, deg0, ts1, agg1, deg1, wa1, ba1, wa2):
    return pl.pallas_call(
        _t3_body,
        grid=(NP // BM,),
        in_specs=_EIN + _EIN +
                 [_full((OUT, HID)), _full((1, HID)), _full((1, HID))],
        out_specs=pl.BlockSpec((8, 128), lambda i: (0, 0)),
        out_shape=jax.ShapeDtypeStruct((8, 128), jnp.float32),
    )(ts0, agg0, deg0, ts1, agg1, deg1,
      wa1, ba1.reshape(1, HID), wa2.reshape(1, HID))


def _t4_body(ts0, agg0, deg0, ts1, agg1, deg1, s_ref, out_o):
    w0 = s_ref[0, 0] / N
    w1 = s_ref[0, 1] / N
    m = jnp.maximum(w0, w1)
    x0 = jnp.exp(w0 - m)
    x1 = jnp.exp(w1 - m)
    beta0 = x0 / (x0 + x1)
    e0 = _elu_norm(ts0, agg0, deg0)
    e1 = _elu_norm(ts1, agg1, deg1)
    out_o[...] = beta0 * e0 + (1.0 - beta0) * e1


def _t4(ts0, agg0, deg0, ts1, agg1, deg1, s):
    return pl.pallas_call(
        _t4_body,
        grid=(NP // BM,),
        in_specs=_EIN + _EIN + [_full((8, 128))],
        out_specs=pl.BlockSpec((BM, OUT), lambda i: (i, 0)),
        out_shape=jax.ShapeDtypeStruct((NP, OUT), jnp.float32),
    )(ts0, agg0, deg0, ts1, agg1, deg1, s)


def kernel(h, edge_index_0, edge_index_1,
           W_self_00, W_neigh_00, b_00, W_self_01, W_neigh_01, b_01,
           W_self_10, W_neigh_10, b_10, W_self_11, W_neigh_11, b_11,
           Wa1, ba1, Wa2):
    src0 = edge_index_0[0].astype(jnp.int32)
    dst0 = edge_index_0[1].astype(jnp.int32)
    src1 = edge_index_1[0].astype(jnp.int32)
    dst1 = edge_index_1[1].astype(jnp.int32)

    hp = jnp.zeros((NP, D), jnp.float32).at[:N].set(h.astype(jnp.float32))

    ts00, tn00, ts10, tn10 = _t1(hp, W_self_00, W_neigh_00, b_00,
                                 W_self_10, W_neigh_10, b_10)
    agg00, deg0 = _segsum(tn00, src0, dst0)
    agg10, deg1 = _segsum(tn10, src1, dst1)

    ts01, tn01 = _t2(ts00, agg00, deg0, W_self_01, W_neigh_01, b_01)
    agg01, _ = _segsum(tn01, src0, dst0)
    ts11, tn11 = _t2(ts10, agg10, deg1, W_self_11, W_neigh_11, b_11)
    agg11, _ = _segsum(tn11, src1, dst1)

    s = _t3(ts01, agg01, deg0, ts11, agg11, deg1, Wa1, ba1, Wa2)
    out = _t4(ts01, agg01, deg0, ts11, agg11, deg1, s)
    return out[:N]


# speedup vs baseline: 1.3637x; 1.0008x over previous
import functools

import jax
import jax.numpy as jnp
from jax import lax
from jax.experimental import pallas as pl
from jax.experimental.pallas import tpu as pltpu
from jax.experimental.pallas import tpu_sc as plsc

N = 10000
NP = 10240
D = 256
OUT = 256
HID = 128
E = 160000

NC = 2
NS = 16
NQ = 4
QR = NP // NQ
RPT = QR // NS
EPT = E // NS
GCH = 128
PADROW = NP - 8
KCAP = EPT + GCH + 64
TRASH = KCAP - 16

BM = 2560


def _sc_segsum_body(xn, srcl, dstl, agg_o, deg_o,
                    src_buf, dst_buf, ksrc, kdst, sidx, didx, rows,
                    degt, zbuf, acc, sdeg, sem):
    c = lax.axis_index("c")
    s = lax.axis_index("s")
    zero16f = jnp.zeros((16,), jnp.float32)
    iota16 = lax.iota(jnp.int32, 16)

    def _zb(i, _):
        def _zl(k, _):
            zbuf[i, 0, pl.ds(k * 16, 16)] = zero16f
            zbuf[i, 1, pl.ds(k * 16, 16)] = zero16f
            return 0
        return lax.fori_loop(0, 8, _zl, 0)
    lax.fori_loop(0, 16, _zb, 0)

    ebase = s * EPT
    pltpu.sync_copy(srcl.at[pl.ds(ebase, EPT)], src_buf)
    pltpu.sync_copy(dstl.at[pl.ds(ebase, EPT)], dst_buf)

    base = s * RPT

    for p in range(2):
        q = 2 * p + c
        lo = q * QR

        def _zd(i, _):
            degt[pl.ds(i * 16, 16)] = zero16f
            return 0
        lax.fori_loop(0, QR // 16, _zd, 0)
        for off in range(0, RPT, 16):
            pltpu.sync_copy(zbuf, acc.at[pl.ds(base + off, 16)])

        @pl.when(s == 0)
        def _():
            pltpu.sync_copy(degt, sdeg)

        plsc.subcore_barrier()

        def _filt(j, cnt):
            sv = src_buf[pl.ds(j * 16, 16)]
            dv = dst_buf[pl.ds(j * 16, 16)]
            dl = dv - lo
            m = (dl >= 0) & (dl < QR)
            mi = jnp.where(m, 1, 0).astype(jnp.int32)
            inc = plsc.cumsum(mi)
            widx = jnp.where(m, cnt + inc - mi, TRASH + iota16)
            plsc.store_scatter(ksrc, [widx], sv)
            plsc.store_scatter(kdst, [widx], dl)
            return cnt + inc[15]
        cnt = lax.fori_loop(0, EPT // 16, _filt, jnp.int32(0))

        pad16 = jnp.full((16,), PADROW, jnp.int32)
        zero16i = jnp.zeros((16,), jnp.int32)
        for k in range(GCH // 16):
            ksrc[pl.ds(cnt + k * 16, 16)] = pad16
            kdst[pl.ds(cnt + k * 16, 16)] = zero16i
        ngc = (cnt + GCH - 1) // GCH

        ones16 = jnp.ones((16,), jnp.float32)

        def _gat(g, last):
            gb = g * GCH
            for k in range(GCH // 16):
                didx[pl.ds(k * 16, 16)] = kdst[pl.ds(gb + k * 16, 16)]
            pltpu.async_copy(
                xn.at[ksrc.at[pl.ds(gb, GCH)]], rows, sem).wait()
            pltpu.sync_copy(rows, acc.at[didx], add=True)
            for k in range(GCH // 16):
                dv = didx[pl.ds(k * 16, 16)]
                if last:
                    ones = jnp.where(gb + k * 16 + iota16 < cnt, 1.0, 0.0)
                else:
                    ones = ones16
                plsc.addupdate_scatter(degt, [dv], ones)

        def _gat_full(g, _):
            _gat(g, False)
            return 0
        lax.fori_loop(0, ngc - 1, _gat_full, 0)

        @pl.when(ngc > 0)
        def _():
            _gat(ngc - 1, True)

        def _dr(g, _):
            gb = g * GCH
            for k in range(GCH // 16):
                didx[pl.ds(k * 16, 16)] = iota16 + (gb + k * 16)
            pltpu.sync_copy(degt.at[pl.ds(gb, GCH)], sdeg.at[didx], add=True)
            return 0
        lax.fori_loop(0, QR // GCH, _dr, 0)

        plsc.subcore_barrier()

        pltpu.sync_copy(acc.at[pl.ds(base, RPT)],
                        agg_o.at[pl.ds(lo + base, RPT)])

        @pl.when(s == 0)
        def _():
            pltpu.sync_copy(sdeg, deg_o.at[pl.ds(lo, QR)])

        plsc.subcore_barrier()


@functools.lru_cache(maxsize=1)
def _build_sc_segsum():
  return functools.partial(
    pl.kernel,
    out_type=[
        jax.ShapeDtypeStruct((NP, 2, OUT // 2), jnp.float32),
        jax.ShapeDtypeStruct((NP,), jnp.float32),
    ],
    mesh=plsc.VectorSubcoreMesh(
        core_axis_name="c", subcore_axis_name="s",
        num_cores=NC, num_subcores=NS),
    compiler_params=pltpu.CompilerParams(needs_layout_passes=False),
    scratch_types=[
        pltpu.VMEM((EPT,), jnp.int32),
        pltpu.VMEM((EPT,), jnp.int32),
        pltpu.VMEM((KCAP,), jnp.int32),
        pltpu.VMEM((KCAP,), jnp.int32),
        pltpu.VMEM((GCH,), jnp.int32),
        pltpu.VMEM((GCH,), jnp.int32),
        pltpu.VMEM((GCH, 2, OUT // 2), jnp.float32),
        pltpu.VMEM((QR,), jnp.float32),
        pltpu.VMEM((16, 2, OUT // 2), jnp.float32),
        pltpu.VMEM_SHARED((QR, 2, OUT // 2), jnp.float32),
        pltpu.VMEM_SHARED((QR,), jnp.float32),
        pltpu.SemaphoreType.DMA,
    ],
  )(_sc_segsum_body)


def _segsum(xn, src, dst):
    agg, deg = _build_sc_segsum()(xn.reshape(NP, 2, OUT // 2), src, dst)
    return agg.reshape(NP, OUT), deg.reshape(NP, 1)


def _dot(a, b):
    return lax.dot_general(a, b, (((1,), (0,)), ((), ())),
                           preferred_element_type=jnp.float32)


def _row_mask(i):
    r = i * BM + lax.broadcasted_iota(jnp.int32, (BM, 1), 0)
    return r < N


def _full(shape):
    return pl.BlockSpec(shape, lambda i: (0,) * len(shape))


_ROW = pl.BlockSpec((BM, D), lambda i: (i, 0))
_DEGB = pl.BlockSpec((BM, 1), lambda i: (i, 0))


def _t1_body(h, ws0, wn0, b0, ws1, wn1, b1, ts0_o, tn0_o, ts1_o, tn1_o):
    x = h[...]
    ts0_o[...] = _dot(x, ws0[...]) + b0[...]
    tn0_o[...] = _dot(x, wn0[...])
    ts1_o[...] = _dot(x, ws1[...]) + b1[...]
    tn1_o[...] = _dot(x, wn1[...])


def _t1(h, ws0, wn0, b0, ws1, wn1, b1):
    return pl.pallas_call(
        _t1_body,
        grid=(NP // BM,),
        in_specs=[_ROW, _full((D, OUT)), _full((D, OUT)), _full((1, OUT)),
                  _full((D, OUT)), _full((D, OUT)), _full((1, OUT))],
        out_specs=[pl.BlockSpec((BM, OUT), lambda i: (i, 0))] * 4,
        out_shape=[jax.ShapeDtypeStruct((NP, OUT), jnp.float32)] * 4,
    )(h, ws0, wn0, b0.reshape(1, OUT), ws1, wn1, b1.reshape(1, OUT))


def _elu_norm(ts_ref, agg_ref, deg_ref):
    rdeg = 1.0 / jnp.maximum(deg_ref[...], 1.0)
    x = ts_ref[...] + agg_ref[...] * rdeg
    return jnp.where(x > 0, x, jnp.exp(jnp.minimum(x, 0.0)) - 1.0)


def _t2_body(ts_ref, agg_ref, deg_ref, ws, wn, b, ts_o, tn_o):
    e = _elu_norm(ts_ref, agg_ref, deg_ref)
    ts_o[...] = _dot(e, ws[...]) + b[...]
    tn_o[...] = jnp.where(_row_mask(pl.program_id(0)), _dot(e, wn[...]), 0.0)


_EIN = [_ROW, pl.BlockSpec((BM, OUT), lambda i: (i, 0)), _DEGB]


def _t2(ts, agg, deg, ws, wn, b):
    return pl.pallas_call(
        _t2_body,
        grid=(NP // BM,),
        in_specs=_EIN + [_full((D, OUT)), _full((D, OUT)), _full((1, OUT))],
        out_specs=[pl.BlockSpec((BM, OUT), lambda i: (i, 0))] * 2,
        out_shape=[jax.ShapeDtypeStruct((NP, OUT), jnp.float32)] * 2,
    )(ts, agg, deg, ws, wn, b.reshape(1, OUT))


def _t3_body(ts0, agg0, deg0, ts1, agg1, deg1, wa1, ba1, wa2, s_o):
    i = pl.program_id(0)

    @pl.when(i == 0)
    def _():
        s_o[...] = jnp.zeros((8, 128), jnp.float32)

    mask = _row_mask(i)
    w = []
    for ts_ref, agg_ref, deg_ref in ((ts0, agg0, deg0), (ts1, agg1, deg1)):
        e = _elu_norm(ts_ref, agg_ref, deg_ref)
        t = jnp.tanh(_dot(e, wa1[...]) + ba1[...])
        w.append(jnp.sum(jnp.where(mask, t * wa2[...], 0.0)))
    r = lax.broadcasted_iota(jnp.int32, (8, 128), 0)
    col = lax.broadcasted_iota(jnp.int32, (8, 128), 1)
    upd = jnp.where((r == 0) & (col == 0), w[0], 0.0) + \
          jnp.where((r == 0) & (col == 1), w[1], 0.0)
    s_o[...] += upd


def _t3(ts0, agg0, deg0, ts1, agg1, deg1, wa1, ba1, wa2):
    return pl.pallas_call(
        _t3_body,
        grid=(NP // BM,),
        in_specs=_EIN + _EIN +
                 [_full((OUT, HID)), _full((1, HID)), _full((1, HID))],
        out_specs=pl.BlockSpec((8, 128), lambda i: (0, 0)),
        out_shape=jax.ShapeDtypeStruct((8, 128), jnp.float32),
    )(ts0, agg0, deg0, ts1, agg1, deg1,
      wa1, ba1.reshape(1, HID), wa2.reshape(1, HID))


def _t4_body(ts0, agg0, deg0, ts1, agg1, deg1, s_ref, out_o):
    w0 = s_ref[0, 0] / N
    w1 = s_ref[0, 1] / N
    m = jnp.maximum(w0, w1)
    x0 = jnp.exp(w0 - m)
    x1 = jnp.exp(w1 - m)
    beta0 = x0 / (x0 + x1)
    e0 = _elu_norm(ts0, agg0, deg0)
    e1 = _elu_norm(ts1, agg1, deg1)
    out_o[...] = beta0 * e0 + (1.0 - beta0) * e1


def _t4(ts0, agg0, deg0, ts1, agg1, deg1, s):
    return pl.pallas_call(
        _t4_body,
        grid=(NP // BM,),
        in_specs=_EIN + _EIN + [_full((8, 128))],
        out_specs=pl.BlockSpec((BM, OUT), lambda i: (i, 0)),
        out_shape=jax.ShapeDtypeStruct((NP, OUT), jnp.float32),
    )(ts0, agg0, deg0, ts1, agg1, deg1, s)


def kernel(h, edge_index_0, edge_index_1,
           W_self_00, W_neigh_00, b_00, W_self_01, W_neigh_01, b_01,
           W_self_10, W_neigh_10, b_10, W_self_11, W_neigh_11, b_11,
           Wa1, ba1, Wa2):
    src0 = edge_index_0[0].astype(jnp.int32)
    dst0 = edge_index_0[1].astype(jnp.int32)
    src1 = edge_index_1[0].astype(jnp.int32)
    dst1 = edge_index_1[1].astype(jnp.int32)

    hp = jnp.zeros((NP, D), jnp.float32).at[:N].set(h.astype(jnp.float32))

    ts00, tn00, ts10, tn10 = _t1(hp, W_self_00, W_neigh_00, b_00,
                                 W_self_10, W_neigh_10, b_10)
    agg00, deg0 = _segsum(tn00, src0, dst0)
    agg10, deg1 = _segsum(tn10, src1, dst1)

    ts01, tn01 = _t2(ts00, agg00, deg0, W_self_01, W_neigh_01, b_01)
    agg01, _ = _segsum(tn01, src0, dst0)
    ts11, tn11 = _t2(ts10, agg10, deg1, W_self_11, W_neigh_11, b_11)
    agg11, _ = _segsum(tn11, src1, dst1)

    s = _t3(ts01, agg01, deg0, ts11, agg11, deg1, Wa1, ba1, Wa2)
    out = _t4(ts01, agg01, deg0, ts11, agg11, deg1, s)
    return out[:N]
